# baseline (speedup 1.0000x reference)
import functools

import jax
import jax.numpy as jnp
from jax import lax
from jax.experimental import pallas as pl
from jax.experimental.pallas import tpu as pltpu
from jax.experimental.pallas import tpu_sc as plsc

_NC = 2
_NSC = 16
_NW = _NC * _NSC
_SC_FRAC = 4
_CH = 128
_TC_STRIPS = 12
_TC_D = 4


def _tc_body(a, p, scale_ref, x_hbm, o_hbm, in_buf, out_buf, in_sems,
             out_sems):
    sr = a // _TC_STRIPS

    def in_copy(s):
        return pltpu.make_async_copy(
            x_hbm.at[0, pl.ds(s * sr, sr), :], in_buf.at[s % _TC_D],
            in_sems.at[s % _TC_D])

    def out_copy(s):
        return pltpu.make_async_copy(
            out_buf.at[s % _TC_D], o_hbm.at[pl.ds(s * sr, sr), :],
            out_sems.at[s % _TC_D])

    scale = scale_ref[0]
    for s in range(_TC_D):
        in_copy(s).start()
    for s in range(_TC_STRIPS):
        slot = s % _TC_D
        in_copy(s).wait()
        if s >= _TC_D:
            out_copy(s - _TC_D).wait()
        e = jnp.exp(in_buf[slot] * scale)
        out_buf[slot] = e / jnp.sum(e, axis=-1, keepdims=True)
        out_copy(s).start()
        if s + _TC_D < _TC_STRIPS:
            in_copy(s + _TC_D).start()
    for s in range(_TC_STRIPS - _TC_D, _TC_STRIPS):
        out_copy(s).wait()


def _lane_sum(v):
    dnums = lax.GatherDimensionNumbers(
        offset_dims=(), collapsed_slice_dims=(0,), start_index_map=(0,))
    lanes = lax.iota(jnp.int32, 16)
    for m in (8, 4, 2, 1):
        perm = lax.gather(v, (lanes ^ m)[:, None], dnums, (1,),
                          mode=lax.GatherScatterMode.PROMISE_IN_BOUNDS)
        v = v + perm
    return v


def _sc_body(a, n, p, scale_hbm, x_hbm, o_hbm, scale_v,
             b0, b1, i0, i1, o0, o1):
    rpw = (n - a) // _NW
    nch = rpw // _CH
    bufs = (b0, b1)
    isems = (i0, i1)
    osems = (o0, o1)
    nd = min(2, nch)
    wid = lax.axis_index("s") * _NC + lax.axis_index("c")
    base = wid * rpw
    pltpu.sync_copy(scale_hbm, scale_v)
    vs = scale_v[...]

    def in_copy(k):
        return pltpu.make_async_copy(
            x_hbm.at[0, pl.ds(a + base + k * _CH, _CH), :], bufs[k % nd],
            isems[k % nd])

    def out_copy(k):
        return pltpu.make_async_copy(
            bufs[k % nd], o_hbm.at[pl.ds(base + k * _CH, _CH), :],
            osems[k % nd])

    def compute(buf):
        def one_row(r):
            e0 = jnp.exp(buf[r, pl.ds(0, 16)] * vs)
            e1 = jnp.exp(buf[r, pl.ds(16, 16)] * vs)
            e2 = jnp.exp(buf[r, pl.ds(32, 16)] * vs)
            e3 = jnp.exp(buf[r, pl.ds(48, 16)] * vs)
            inv = 1.0 / _lane_sum(e0 + e1 + e2 + e3)
            buf[r, pl.ds(0, 16)] = e0 * inv
            buf[r, pl.ds(16, 16)] = e1 * inv
            buf[r, pl.ds(32, 16)] = e2 * inv
            buf[r, pl.ds(48, 16)] = e3 * inv

        def rows(i, c):
            r = i * 4
            one_row(r)
            one_row(r + 1)
            one_row(r + 2)
            one_row(r + 3)
            return c

        lax.fori_loop(0, _CH // 4, rows, 0)

    for k in range(nd):
        in_copy(k).start()
    for k in range(nch):
        in_copy(k).wait()
        if k >= nd:
            out_copy(k - nd).wait()
        compute(bufs[k % nd])
        out_copy(k).start()
        if k + nd < nch:
            in_copy(k + nd).start()
    for k in range(max(0, nch - nd), nch):
        out_copy(k).wait()


def kernel(branch, par, chi, t):
    _, n, p = branch.shape
    a = n - n // _SC_FRAC
    sr = a // _TC_STRIPS
    scale_s = (2.0 / jnp.asarray(t, jnp.float32)).reshape(1)
    scale_v = jnp.full((16,), 2.0 / jnp.asarray(t, jnp.float32), jnp.float32)

    tc_full = pl.pallas_call(
        functools.partial(_tc_body, a, p),
        in_specs=[
            pl.BlockSpec(memory_space=pltpu.SMEM),
            pl.BlockSpec(memory_space=pl.ANY),
        ],
        out_specs=pl.BlockSpec(memory_space=pl.ANY),
        out_shape=jax.ShapeDtypeStruct((n, p), jnp.float32),
        scratch_shapes=[
            pltpu.VMEM((_TC_D, sr, p), jnp.float32),
            pltpu.VMEM((_TC_D, sr, p), jnp.float32),
            pltpu.SemaphoreType.DMA((_TC_D,)),
            pltpu.SemaphoreType.DMA((_TC_D,)),
        ],
    )(scale_s, branch)

    mesh = plsc.VectorSubcoreMesh(
        core_axis_name="c", subcore_axis_name="s",
        num_cores=_NC, num_subcores=_NSC)
    sc_part = pl.kernel(
        functools.partial(_sc_body, a, n, p),
        out_type=jax.ShapeDtypeStruct((n - a, p), jnp.float32),
        mesh=mesh,
        scratch_types=[pltpu.VMEM((16,), jnp.float32)]
        + [pltpu.VMEM((_CH, p), jnp.float32) for _ in range(2)]
        + [pltpu.SemaphoreType.DMA for _ in range(4)],
    )(scale_v, branch)

    return lax.dynamic_update_slice(tc_full, sc_part, (a, 0))
